# baseline (speedup 1.0000x reference)
import functools

import jax
import jax.numpy as jnp
from jax import lax
from jax.experimental import pallas as pl
from jax.experimental.pallas import tpu as pltpu
from jax.experimental.pallas import tpu_sc as plsc

NNODE = 10000
NEDGE = 320000
NPAD = 10240
KPOOL = 5000
KPAD = 5120
DBLK = 128

NC = 2
NS = 16
NW = NC * NS
EPW = NEDGE // NW
ECHUNK = 80
NITER = EPW // ECHUNK


def _make_sc_spmm(spad, npad):
    del spad
    rpt = npad // NS
    mesh = plsc.VectorSubcoreMesh(core_axis_name="c", subcore_axis_name="s")

    @functools.partial(
        pl.kernel,
        mesh=mesh,
        out_type=jax.ShapeDtypeStruct((2 * npad, DBLK), jnp.float32),
        scratch_types=[
            pltpu.VMEM_SHARED((npad, DBLK), jnp.float32),
            pltpu.VMEM((ECHUNK,), jnp.int32),
            pltpu.VMEM((ECHUNK,), jnp.int32),
            pltpu.VMEM((ECHUNK, DBLK), jnp.float32),
            pltpu.SemaphoreType.DMA,
        ],
    )
    def spmm(xa, rows, cols, zz, out, sh, ridx, cidx, gbuf, sem):
        c = lax.axis_index("c")
        s = lax.axis_index("s")
        wid = s * NC + c
        pltpu.sync_copy(zz.at[pl.ds(s * rpt, rpt)], sh.at[pl.ds(s * rpt, rpt)])
        plsc.subcore_barrier()
        ebase = wid * EPW

        def body(i, carry):
            off = pl.multiple_of(ebase + i * ECHUNK, 8)
            pltpu.sync_copy(rows.at[pl.ds(off, ECHUNK)], ridx)
            pltpu.sync_copy(cols.at[pl.ds(off, ECHUNK)], cidx)
            pltpu.async_copy(xa.at[cidx], gbuf, sem).wait()
            pltpu.sync_copy(gbuf, sh.at[ridx], add=True)
            return carry

        lax.fori_loop(0, NITER, body, 0)
        plsc.subcore_barrier()
        obase = pl.multiple_of(c * npad + s * rpt, 8)
        pltpu.sync_copy(sh.at[pl.ds(s * rpt, rpt)], out.at[pl.ds(obase, rpt)])

    return spmm


def _make_sc_deg(npad):
    rpt = npad // NS
    mesh = plsc.VectorSubcoreMesh(core_axis_name="c", subcore_axis_name="s")

    @functools.partial(
        pl.kernel,
        mesh=mesh,
        out_type=jax.ShapeDtypeStruct((2 * npad, DBLK), jnp.float32),
        scratch_types=[
            pltpu.VMEM_SHARED((npad, DBLK), jnp.float32),
            pltpu.VMEM((ECHUNK,), jnp.int32),
            pltpu.VMEM((ECHUNK, DBLK), jnp.float32),
        ],
    )
    def deg(rows, zz, out, sh, ridx, gbuf):
        c = lax.axis_index("c")
        s = lax.axis_index("s")
        wid = s * NC + c
        pltpu.sync_copy(zz.at[pl.ds(s * rpt, rpt)], sh.at[pl.ds(s * rpt, rpt)])
        pltpu.sync_copy(zz.at[pl.ds(0, ECHUNK)], gbuf)
        e0 = jnp.where(lax.iota(jnp.int32, 16) == 0,
                       jnp.float32(1.0), jnp.float32(0.0))

        def fill(i, carry):
            gbuf[i, pl.ds(0, 16)] = e0
            return carry

        lax.fori_loop(0, ECHUNK, fill, 0)
        plsc.subcore_barrier()
        ebase = wid * EPW

        def body(i, carry):
            off = pl.multiple_of(ebase + i * ECHUNK, 8)
            pltpu.sync_copy(rows.at[pl.ds(off, ECHUNK)], ridx)
            pltpu.sync_copy(gbuf, sh.at[ridx], add=True)
            return carry

        lax.fori_loop(0, NITER, body, 0)
        plsc.subcore_barrier()
        obase = pl.multiple_of(c * npad + s * rpt, 8)
        pltpu.sync_copy(sh.at[pl.ds(s * rpt, rpt)], out.at[pl.ds(obase, rpt)])

    return deg


_sc_spmm_n = _make_sc_spmm(NPAD, NPAD)
_sc_spmm_k = _make_sc_spmm(KPAD, KPAD)
_sc_deg_n = _make_sc_deg(NPAD)
_sc_deg_k = _make_sc_deg(KPAD)


def _spmm_full(feats, row, col, spad, npad, n_src, n_dst, sc_fn):
    d = feats.shape[1]
    zz = jnp.zeros((npad, DBLK), jnp.float32)
    halves = []
    for h0 in range(0, d, DBLK):
        xa = jnp.zeros((spad, DBLK), jnp.float32)
        xa = xa.at[:n_src].set(feats[:, h0:h0 + DBLK])
        o2 = sc_fn(xa, row, col, zz)
        halves.append((o2[:npad] + o2[npad:])[:n_dst])
    return jnp.concatenate(halves, axis=1) if len(halves) > 1 else halves[0]


def _degree(row, npad, n_dst, deg_fn):
    zz = jnp.zeros((npad, DBLK), jnp.float32)
    o2 = deg_fn(row, zz)
    return (o2[:npad, 0] + o2[npad:, 0])[:n_dst]


def _mm_body(x_ref, w_ref, o_ref):
    o_ref[...] = jnp.dot(x_ref[...], w_ref[...],
                         preferred_element_type=jnp.float32)


def _matmul(x, w, bm=1000):
    m, k = x.shape
    n = w.shape[1]
    return pl.pallas_call(
        _mm_body,
        grid=(m // bm,),
        in_specs=[
            pl.BlockSpec((bm, k), lambda i: (i, 0)),
            pl.BlockSpec((k, n), lambda i: (0, 0)),
        ],
        out_specs=pl.BlockSpec((bm, n), lambda i: (i, 0)),
        out_shape=jax.ShapeDtypeStruct((m, n), jnp.float32),
    )(x, w)


def kernel(x, edge_index, W0, b0, W1, b1, pw, pb, Wu, bu, Wf, bf):
    row = edge_index[0]
    col = edge_index[1]

    x0 = _matmul(x, W0) + b0
    deg = (_degree(row, NPAD, NNODE, _sc_deg_n) + 1.0)[:, None]
    s1 = _spmm_full(x0, row, col, NPAD, NPAD, NNODE, NNODE, _sc_spmm_n)
    h = jax.nn.relu((s1 + x0) / deg)

    pwp = jnp.zeros((pw.shape[0], 128), jnp.float32).at[:, 0].set(pw[:, 0])
    score_col = _matmul(h, pwp)[:, 0] + pb[0]
    score = score_col / jnp.sqrt(jnp.sum(score_col ** 2))
    score = jax.nn.sigmoid(score)
    vals, idx = jax.lax.top_k(score, KPOOL)
    hp = jnp.take(h, idx, axis=0) * vals[:, None]

    xp = _matmul(hp, W1, bm=1000) + b1
    node_map = jnp.full((NNODE,), -1, jnp.int32).at[idx].set(
        jnp.arange(KPOOL, dtype=jnp.int32))
    r2 = node_map[row]
    c2 = node_map[col]
    valid = (r2 >= 0) & (c2 >= 0)
    rp = jnp.where(valid, r2, KPAD - 1)
    cp = jnp.where(valid, c2, 0)
    degp = (_degree(rp, KPAD, KPOOL, _sc_deg_k) + 1.0)[:, None]
    s2 = _spmm_full(xp, rp, cp, KPAD, KPAD, KPOOL, KPOOL, _sc_spmm_k)
    h2 = jax.nn.relu((s2 + xp) / degp)

    g = _matmul(h2, Wu, bm=1000) + bu
    m3 = jnp.broadcast_to(bu, (NNODE, Wu.shape[1])).at[idx].set(g)
    s3 = _spmm_full(m3, row, col, NPAD, NPAD, NNODE, NNODE, _sc_spmm_n)
    h3 = jax.nn.relu((s3 + m3) / deg)

    m4 = _matmul(h3, Wf) + bf
    s4 = _spmm_full(m4, row, col, NPAD, NPAD, NNODE, NNODE, _sc_spmm_n)
    h4 = (s4 + m4) / deg
    return jax.nn.log_softmax(h4, axis=1)


# speedup vs baseline: 1.0587x; 1.0587x over previous
import functools

import jax
import jax.numpy as jnp
from jax import lax
from jax.experimental import pallas as pl
from jax.experimental.pallas import tpu as pltpu
from jax.experimental.pallas import tpu_sc as plsc

NNODE = 10000
NEDGE = 320000
NPAD = 10240
KPOOL = 5000
KPAD = 5120
DBLK = 128

NC = 2
NS = 16
NW = NC * NS
EPW = NEDGE // NW
ECHUNK = 100
NITER = EPW // ECHUNK
UNROLL = 1
NJ = NITER // UNROLL


def _make_sc_spmm(spad, npad):
    del spad
    rpt = npad // NS
    mesh = plsc.VectorSubcoreMesh(core_axis_name="c", subcore_axis_name="s")

    @functools.partial(
        pl.kernel,
        mesh=mesh,
        out_type=jax.ShapeDtypeStruct((2 * npad, DBLK), jnp.float32),
        scratch_types=[
            pltpu.VMEM_SHARED((npad, DBLK), jnp.float32),
            pltpu.VMEM((NITER, ECHUNK), jnp.int32),
            pltpu.VMEM((NITER, ECHUNK), jnp.int32),
        ] + [pltpu.VMEM((ECHUNK, DBLK), jnp.float32) for _ in range(UNROLL)]
          + [pltpu.SemaphoreType.DMA for _ in range(2 * UNROLL)],
    )
    def spmm(xa, rows3, cols3, zz, out, sh, ridx, cidx, *bufs):
        gbufs = bufs[:UNROLL]
        gsems = bufs[UNROLL:2 * UNROLL]
        ssems = bufs[2 * UNROLL:]
        c = lax.axis_index("c")
        s = lax.axis_index("s")
        wid = s * NC + c
        pltpu.sync_copy(rows3.at[wid], ridx)
        pltpu.sync_copy(cols3.at[wid], cidx)
        pltpu.sync_copy(zz.at[pl.ds(s * rpt, rpt)], sh.at[pl.ds(s * rpt, rpt)])
        plsc.subcore_barrier()

        def body(j, carry):
            base_i = j * UNROLL
            gh = [pltpu.async_copy(xa.at[cidx.at[base_i + k]], gbufs[k],
                                   gsems[k]) for k in range(UNROLL)]
            sh_h = []
            for k in range(UNROLL):
                gh[k].wait()
                sh_h.append(pltpu.async_copy(gbufs[k],
                                             sh.at[ridx.at[base_i + k]],
                                             ssems[k], add=True))
            for k in range(UNROLL):
                sh_h[k].wait()
            return carry

        lax.fori_loop(0, NJ, body, 0)
        plsc.subcore_barrier()
        obase = pl.multiple_of(c * npad + s * rpt, 8)
        pltpu.sync_copy(sh.at[pl.ds(s * rpt, rpt)], out.at[pl.ds(obase, rpt)])

    return spmm


def _make_sc_deg(npad):
    rpt = npad // NS
    mesh = plsc.VectorSubcoreMesh(core_axis_name="c", subcore_axis_name="s")

    @functools.partial(
        pl.kernel,
        mesh=mesh,
        out_type=jax.ShapeDtypeStruct((2 * npad, DBLK), jnp.float32),
        scratch_types=[
            pltpu.VMEM_SHARED((npad, DBLK), jnp.float32),
            pltpu.VMEM((NITER, ECHUNK), jnp.int32),
            pltpu.VMEM((ECHUNK, DBLK), jnp.float32),
        ] + [pltpu.SemaphoreType.DMA for _ in range(UNROLL)],
    )
    def deg(rows3, zz, out, sh, ridx, gbuf, *ssems):
        c = lax.axis_index("c")
        s = lax.axis_index("s")
        wid = s * NC + c
        pltpu.sync_copy(rows3.at[wid], ridx)
        pltpu.sync_copy(zz.at[pl.ds(s * rpt, rpt)], sh.at[pl.ds(s * rpt, rpt)])
        e0 = jnp.where(lax.iota(jnp.int32, 16) == 0,
                       jnp.float32(1.0), jnp.float32(0.0))
        zv = jnp.zeros((16,), jnp.float32)

        def fill(i, carry):
            for cb in range(16, DBLK, 16):
                gbuf[i, pl.ds(cb, 16)] = zv
            gbuf[i, pl.ds(0, 16)] = e0
            return carry

        lax.fori_loop(0, ECHUNK, fill, 0)
        plsc.subcore_barrier()

        def body(j, carry):
            base_i = j * UNROLL
            hs = [pltpu.async_copy(gbuf, sh.at[ridx.at[base_i + k]],
                                   ssems[k], add=True)
                  for k in range(UNROLL)]
            for h in hs:
                h.wait()
            return carry

        lax.fori_loop(0, NJ, body, 0)
        plsc.subcore_barrier()
        obase = pl.multiple_of(c * npad + s * rpt, 8)
        pltpu.sync_copy(sh.at[pl.ds(s * rpt, rpt)], out.at[pl.ds(obase, rpt)])

    return deg


_sc_spmm_n = _make_sc_spmm(NPAD, NPAD)
_sc_spmm_k = _make_sc_spmm(KPAD, KPAD)
_sc_deg_n = _make_sc_deg(NPAD)
_sc_deg_k = _make_sc_deg(KPAD)


def _spmm_full(feats, row, col, spad, npad, n_src, n_dst, sc_fn):
    d = feats.shape[1]
    zz = jnp.zeros((npad, DBLK), jnp.float32)
    rows3 = row.reshape(NW, NITER, ECHUNK)
    cols3 = col.reshape(NW, NITER, ECHUNK)
    halves = []
    for h0 in range(0, d, DBLK):
        xa = jnp.zeros((spad, DBLK), jnp.float32)
        xa = xa.at[:n_src].set(feats[:, h0:h0 + DBLK])
        o2 = sc_fn(xa, rows3, cols3, zz)
        halves.append((o2[:npad] + o2[npad:])[:n_dst])
    return jnp.concatenate(halves, axis=1) if len(halves) > 1 else halves[0]


def _degree(row, npad, n_dst, deg_fn):
    zz = jnp.zeros((npad, DBLK), jnp.float32)
    o2 = deg_fn(row.reshape(NW, NITER, ECHUNK), zz)
    return (o2[:npad, 0] + o2[npad:, 0])[:n_dst]


def _mm_body(x_ref, w_ref, o_ref):
    o_ref[...] = jnp.dot(x_ref[...], w_ref[...],
                         preferred_element_type=jnp.float32)


def _matmul(x, w, bm=1000):
    m, k = x.shape
    n = w.shape[1]
    return pl.pallas_call(
        _mm_body,
        grid=(m // bm,),
        in_specs=[
            pl.BlockSpec((bm, k), lambda i: (i, 0)),
            pl.BlockSpec((k, n), lambda i: (0, 0)),
        ],
        out_specs=pl.BlockSpec((bm, n), lambda i: (i, 0)),
        out_shape=jax.ShapeDtypeStruct((m, n), jnp.float32),
    )(x, w)


def kernel(x, edge_index, W0, b0, W1, b1, pw, pb, Wu, bu, Wf, bf):
    row = edge_index[0]
    col = edge_index[1]

    x0 = _matmul(x, W0) + b0
    deg = (_degree(row, NPAD, NNODE, _sc_deg_n) + 1.0)[:, None]
    s1 = _spmm_full(x0, row, col, NPAD, NPAD, NNODE, NNODE, _sc_spmm_n)
    h = jax.nn.relu((s1 + x0) / deg)

    pwp = jnp.zeros((pw.shape[0], 128), jnp.float32).at[:, 0].set(pw[:, 0])
    score_col = _matmul(h, pwp)[:, 0] + pb[0]
    score = score_col / jnp.sqrt(jnp.sum(score_col ** 2))
    score = jax.nn.sigmoid(score)
    vals, idx = jax.lax.top_k(score, KPOOL)
    hp = jnp.take(h, idx, axis=0) * vals[:, None]

    xp = _matmul(hp, W1, bm=1000) + b1
    node_map = jnp.full((NNODE,), -1, jnp.int32).at[idx].set(
        jnp.arange(KPOOL, dtype=jnp.int32))
    r2 = node_map[row]
    c2 = node_map[col]
    valid = (r2 >= 0) & (c2 >= 0)
    rp = jnp.where(valid, r2, KPAD - 1)
    cp = jnp.where(valid, c2, 0)
    degp = (_degree(rp, KPAD, KPOOL, _sc_deg_k) + 1.0)[:, None]
    s2 = _spmm_full(xp, rp, cp, KPAD, KPAD, KPOOL, KPOOL, _sc_spmm_k)
    h2 = jax.nn.relu((s2 + xp) / degp)

    g = _matmul(h2, Wu, bm=1000) + bu
    m3 = jnp.broadcast_to(bu, (NNODE, Wu.shape[1])).at[idx].set(g)
    s3 = _spmm_full(m3, row, col, NPAD, NPAD, NNODE, NNODE, _sc_spmm_n)
    h3 = jax.nn.relu((s3 + m3) / deg)

    m4 = _matmul(h3, Wf) + bf
    s4 = _spmm_full(m4, row, col, NPAD, NPAD, NNODE, NNODE, _sc_spmm_n)
    h4 = (s4 + m4) / deg
    return jax.nn.log_softmax(h4, axis=1)
